# baseline (speedup 1.0000x reference)
import functools

import jax
import jax.numpy as jnp
from jax import lax
from jax.experimental import pallas as pl
from jax.experimental.pallas import tpu as pltpu
from jax.experimental.pallas import tpu_sc as plsc

_OP_EQ, _OP_LT, _OP_GT = 0, 1, 2
_ALPHA, _BETA = 1.2, 0.7
_B, _D = 8192, 2048

_SC_ROWS = 2048
_TC_ROWS = _B - _SC_ROWS

_BM = 512
_NB = _TC_ROWS // _BM
_L = 128
_TR = 64
_NK = _D // _L
_NT = _BM // _TR

_SC_NC = 2
_SC_NS = 16
_NW = _SC_NC * _SC_NS
_RPW = _SC_ROWS // _NW
_CH = 8
_NCH = _RPW // _CH
_V = 16


def _tc_body(op_full_ref, op_ref, j1_ref, j2_ref, out_ref, acc_ref):
    i = pl.program_id(0)

    @pl.when(i == 0)
    def _init():
        acc_ref[...] = jnp.zeros((3, _TR, _L), jnp.float32)

    ones_b = jnp.ones((_L, _L), dtype=jnp.bfloat16)
    acc0 = acc_ref[0]
    acc1 = acc_ref[1]
    acc2 = acc_ref[2]
    for r in range(_NT):
        r0 = r * _TR
        a = j1_ref[r0:r0 + _TR, 0:_L].astype(jnp.bfloat16)
        b = j2_ref[r0:r0 + _TR, 0:_L].astype(jnp.bfloat16)
        p1 = a * a
        p2 = b * b
        p12 = a * b
        for k in range(1, _NK):
            c0 = k * _L
            a = j1_ref[r0:r0 + _TR, c0:c0 + _L].astype(jnp.bfloat16)
            b = j2_ref[r0:r0 + _TR, c0:c0 + _L].astype(jnp.bfloat16)
            p1 += a * a
            p2 += b * b
            p12 += a * b
        s1 = jax.lax.dot(p1, ones_b, preferred_element_type=jnp.float32)
        s2 = jax.lax.dot(p2, ones_b, preferred_element_type=jnp.float32)
        pd = (p1 + p2 - 2.0 * p12).astype(jnp.float32)
        op_t = op_ref[r0:r0 + _TR, :]
        eq = (op_t == _OP_EQ).astype(jnp.float32)
        dn = jnp.sqrt(s1) - jnp.sqrt(s2)
        acc0 = acc0 + eq * pd
        acc1 = acc1 + jnp.maximum(dn, 0.0)
        acc2 = acc2 + jnp.maximum(-dn, 0.0)
    acc_ref[0] = acc0
    acc_ref[1] = acc1
    acc_ref[2] = acc2

    @pl.when(i == _NB - 1)
    def _finalize():
        inv_l = 1.0 / _L
        opf = op_full_ref[0, :]
        out_ref[0, 0] = jnp.sum(acc_ref[0])
        out_ref[0, 1] = jnp.sum(acc_ref[1]) * inv_l
        out_ref[0, 2] = jnp.sum(acc_ref[2]) * inv_l
        out_ref[0, 3] = jnp.sum((opf == _OP_EQ).astype(jnp.float32))
        out_ref[0, 4] = jnp.sum((opf == _OP_LT).astype(jnp.float32))
        out_ref[0, 5] = jnp.sum((opf == _OP_GT).astype(jnp.float32))


def _tc_partials(joint1_embedding, joint2_embedding, operation, op_row):
    return pl.pallas_call(
        _tc_body,
        grid=(_NB,),
        in_specs=[
            pl.BlockSpec((1, _B), lambda i: (0, 0)),
            pl.BlockSpec((_BM, 1), lambda i: (i, 0)),
            pl.BlockSpec((_BM, _D), lambda i: (i, 0)),
            pl.BlockSpec((_BM, _D), lambda i: (i, 0)),
        ],
        out_specs=pl.BlockSpec(memory_space=pltpu.SMEM),
        out_shape=jax.ShapeDtypeStruct((1, 8), jnp.float32),
        scratch_shapes=[
            pltpu.VMEM((3, _TR, _L), jnp.float32),
        ],
    )(op_row, operation, joint1_embedding, joint2_embedding)


def _sc_rowsums(joint1_embedding, joint2_embedding):
    mesh = plsc.VectorSubcoreMesh(core_axis_name="c", subcore_axis_name="s")

    @functools.partial(
        pl.kernel,
        mesh=mesh,
        out_type=jax.ShapeDtypeStruct((3 * _SC_ROWS * _V,), jnp.float32),
        scratch_types=[
            pltpu.VMEM((2, _CH, _D), jnp.float32),
            pltpu.VMEM((2, _CH, _D), jnp.float32),
            pltpu.VMEM((3 * _RPW * _V,), jnp.float32),
            pltpu.SemaphoreType.DMA,
            pltpu.SemaphoreType.DMA,
            pltpu.SemaphoreType.DMA,
            pltpu.SemaphoreType.DMA,
        ],
    )
    def _sc_kernel(j1_hbm, j2_hbm, out_hbm, buf1, buf2, res,
                   s1a, s1b, s2a, s2b):
        c = lax.axis_index("c")
        s = lax.axis_index("s")
        wid = s * _SC_NC + c
        base = _TC_ROWS + wid * _RPW
        sems1 = (s1a, s1b)
        sems2 = (s2a, s2b)

        def _start(chunk):
            slot = chunk % 2
            r0 = base + chunk * _CH
            pltpu.make_async_copy(
                j1_hbm.at[pl.ds(r0, _CH), :], buf1.at[slot], sems1[slot]
            ).start()
            pltpu.make_async_copy(
                j2_hbm.at[pl.ds(r0, _CH), :], buf2.at[slot], sems2[slot]
            ).start()

        def _wait(chunk):
            slot = chunk % 2
            r0 = base + chunk * _CH
            pltpu.make_async_copy(
                j1_hbm.at[pl.ds(r0, _CH), :], buf1.at[slot], sems1[slot]
            ).wait()
            pltpu.make_async_copy(
                j2_hbm.at[pl.ds(r0, _CH), :], buf2.at[slot], sems2[slot]
            ).wait()

        _start(0)
        for chunk in range(_NCH):
            if chunk + 1 < _NCH:
                _start(chunk + 1)
            _wait(chunk)
            slot = chunk % 2
            for r in range(_CH):
                zero = jnp.zeros((_V,), jnp.float32)

                def _kbody(k, accs, _slot=slot, _r=r):
                    a1, a2, a12 = accs
                    col = k * (_V * 4)
                    for u in range(4):
                        off = col + u * _V
                        va = buf1[_slot, _r, pl.ds(off, _V)]
                        vb = buf2[_slot, _r, pl.ds(off, _V)]
                        a1 = a1 + va * va
                        a2 = a2 + vb * vb
                        a12 = a12 + va * vb
                    return (a1, a2, a12)

                acc1, acc2, acc12 = lax.fori_loop(
                    0, _D // (_V * 4), _kbody, (zero, zero, zero))
                rv = (chunk * _CH + r) * _V
                res[pl.ds(rv, _V)] = acc1
                res[pl.ds(_RPW * _V + rv, _V)] = acc2
                res[pl.ds(2 * _RPW * _V + rv, _V)] = acc12
        for q in range(3):
            pltpu.sync_copy(
                res.at[pl.ds(q * _RPW * _V, _RPW * _V)],
                out_hbm.at[pl.ds(q * _SC_ROWS * _V + wid * _RPW * _V,
                                 _RPW * _V)])

    return _sc_kernel(joint1_embedding, joint2_embedding)


def _fin_body(tc_ref, s1_ref, s2_ref, s12_ref, op_ref, out_ref):
    s1 = jnp.sum(s1_ref[...], axis=1)
    s2 = jnp.sum(s2_ref[...], axis=1)
    s12 = jnp.sum(s12_ref[...], axis=1)
    op_t = op_ref[0, _TC_ROWS:_B]
    eq = (op_t == _OP_EQ).astype(jnp.float32)
    sd = s1 + s2 - 2.0 * s12
    dn = jnp.sqrt(s1) - jnp.sqrt(s2)
    eq_sd = tc_ref[0, 0] + jnp.sum(eq * sd)
    lt_sum = tc_ref[0, 1] + jnp.sum(jnp.maximum(dn, 0.0))
    gt_sum = tc_ref[0, 2] + jnp.sum(jnp.maximum(-dn, 0.0))
    eq_cnt = tc_ref[0, 3]
    has_lt = (tc_ref[0, 4] > 0.0).astype(jnp.float32)
    has_gt = (tc_ref[0, 5] > 0.0).astype(jnp.float32)
    eq_loss = eq_sd / jnp.maximum(eq_cnt * _D, 1.0)
    lt_loss = lt_sum * (1.0 / _B)
    gt_loss = gt_sum * (1.0 / _B)
    out_ref[0, 0] = (_ALPHA * eq_loss
                     + _BETA * (has_lt * lt_loss + has_gt * gt_loss))


def kernel(joint1_embedding, joint2_embedding, operation):
    op_row = operation.reshape(1, _B)
    tc_parts = _tc_partials(joint1_embedding, joint2_embedding, operation,
                            op_row)
    sc_parts = _sc_rowsums(joint1_embedding, joint2_embedding)
    sc_parts = sc_parts.reshape(3, _SC_ROWS, _V)
    out = pl.pallas_call(
        _fin_body,
        in_specs=[
            pl.BlockSpec(memory_space=pltpu.SMEM),
            pl.BlockSpec((_SC_ROWS, _V), lambda: (0, 0)),
            pl.BlockSpec((_SC_ROWS, _V), lambda: (0, 0)),
            pl.BlockSpec((_SC_ROWS, _V), lambda: (0, 0)),
            pl.BlockSpec((1, _B), lambda: (0, 0)),
        ],
        out_specs=pl.BlockSpec(memory_space=pltpu.SMEM),
        out_shape=jax.ShapeDtypeStruct((1, 1), jnp.float32),
    )(tc_parts, sc_parts[0], sc_parts[1], sc_parts[2], op_row)
    return out[0, 0]


# speedup vs baseline: 1.0164x; 1.0164x over previous
import functools

import jax
import jax.numpy as jnp
from jax import lax
from jax.experimental import pallas as pl
from jax.experimental.pallas import tpu as pltpu
from jax.experimental.pallas import tpu_sc as plsc

_OP_EQ, _OP_LT, _OP_GT = 0, 1, 2
_ALPHA, _BETA = 1.2, 0.7
_B, _D = 8192, 2048

_SC_ROWS = 2048
_TC_ROWS = _B - _SC_ROWS

_BM = 512
_NB = _TC_ROWS // _BM
_L = 128
_TR = 64
_NK = _D // _L
_NT = _BM // _TR

_SC_NC = 2
_SC_NS = 16
_NW = _SC_NC * _SC_NS
_RPW = _SC_ROWS // _NW
_CH = 8
_NCH = _RPW // _CH
_V = 16


def _tc_body(op_full_ref, op_ref, j1_ref, j2_ref, out_ref, acc_ref):
    i = pl.program_id(0)

    @pl.when(i == 0)
    def _init():
        acc_ref[...] = jnp.zeros((3, _TR, _L), jnp.float32)

    ones_b = jnp.ones((_L, _L), dtype=jnp.bfloat16)
    acc0 = acc_ref[0]
    acc1 = acc_ref[1]
    acc2 = acc_ref[2]
    for r in range(_NT):
        r0 = r * _TR
        a = j1_ref[r0:r0 + _TR, 0:_L].astype(jnp.bfloat16)
        b = j2_ref[r0:r0 + _TR, 0:_L].astype(jnp.bfloat16)
        p1 = a * a
        p2 = b * b
        p12 = a * b
        for k in range(1, _NK):
            c0 = k * _L
            a = j1_ref[r0:r0 + _TR, c0:c0 + _L].astype(jnp.bfloat16)
            b = j2_ref[r0:r0 + _TR, c0:c0 + _L].astype(jnp.bfloat16)
            p1 += a * a
            p2 += b * b
            p12 += a * b
        s1 = jax.lax.dot(p1, ones_b, preferred_element_type=jnp.float32)
        s2 = jax.lax.dot(p2, ones_b, preferred_element_type=jnp.float32)
        pd = (p1 + p2 - 2.0 * p12).astype(jnp.float32)
        op_t = op_ref[r0:r0 + _TR, :]
        eq = (op_t == _OP_EQ).astype(jnp.float32)
        dn = jnp.sqrt(s1) - jnp.sqrt(s2)
        acc0 = acc0 + eq * pd
        acc1 = acc1 + jnp.maximum(dn, 0.0)
        acc2 = acc2 + jnp.maximum(-dn, 0.0)
    acc_ref[0] = acc0
    acc_ref[1] = acc1
    acc_ref[2] = acc2

    @pl.when(i == _NB - 1)
    def _finalize():
        inv_l = 1.0 / _L
        opf = op_full_ref[0, :]
        out_ref[0, 0] = jnp.sum(acc_ref[0])
        out_ref[0, 1] = jnp.sum(acc_ref[1]) * inv_l
        out_ref[0, 2] = jnp.sum(acc_ref[2]) * inv_l
        out_ref[0, 3] = jnp.sum((opf == _OP_EQ).astype(jnp.float32))
        out_ref[0, 4] = jnp.sum((opf == _OP_LT).astype(jnp.float32))
        out_ref[0, 5] = jnp.sum((opf == _OP_GT).astype(jnp.float32))


def _tc_partials(joint1_embedding, joint2_embedding, operation, op_row):
    return pl.pallas_call(
        _tc_body,
        grid=(_NB,),
        in_specs=[
            pl.BlockSpec((1, _B), lambda i: (0, 0)),
            pl.BlockSpec((_BM, 1), lambda i: (i, 0)),
            pl.BlockSpec((_BM, _D), lambda i: (i, 0)),
            pl.BlockSpec((_BM, _D), lambda i: (i, 0)),
        ],
        out_specs=pl.BlockSpec(memory_space=pltpu.SMEM),
        out_shape=jax.ShapeDtypeStruct((1, 8), jnp.float32),
        scratch_shapes=[
            pltpu.VMEM((3, _TR, _L), jnp.float32),
        ],
    )(op_row, operation, joint1_embedding, joint2_embedding)


def _sc_rowsums(joint1_embedding, joint2_embedding):
    mesh = plsc.VectorSubcoreMesh(core_axis_name="c", subcore_axis_name="s")

    @functools.partial(
        pl.kernel,
        mesh=mesh,
        out_type=jax.ShapeDtypeStruct((3 * _SC_ROWS * _V,), jnp.float32),
        scratch_types=[
            pltpu.VMEM((2, _CH, _D), jnp.float32),
            pltpu.VMEM((2, _CH, _D), jnp.float32),
            pltpu.VMEM((3 * _RPW * _V,), jnp.float32),
            pltpu.SemaphoreType.DMA,
            pltpu.SemaphoreType.DMA,
            pltpu.SemaphoreType.DMA,
            pltpu.SemaphoreType.DMA,
        ],
    )
    def _sc_kernel(j1_hbm, j2_hbm, out_hbm, buf1, buf2, res,
                   s1a, s1b, s2a, s2b):
        c = lax.axis_index("c")
        s = lax.axis_index("s")
        wid = s * _SC_NC + c
        base = _TC_ROWS + wid * _RPW
        sems1 = (s1a, s1b)
        sems2 = (s2a, s2b)

        def _start(chunk):
            slot = chunk % 2
            r0 = base + chunk * _CH
            pltpu.make_async_copy(
                j1_hbm.at[pl.ds(r0, _CH), :], buf1.at[slot], sems1[slot]
            ).start()
            pltpu.make_async_copy(
                j2_hbm.at[pl.ds(r0, _CH), :], buf2.at[slot], sems2[slot]
            ).start()

        def _wait(chunk):
            slot = chunk % 2
            r0 = base + chunk * _CH
            pltpu.make_async_copy(
                j1_hbm.at[pl.ds(r0, _CH), :], buf1.at[slot], sems1[slot]
            ).wait()
            pltpu.make_async_copy(
                j2_hbm.at[pl.ds(r0, _CH), :], buf2.at[slot], sems2[slot]
            ).wait()

        _start(0)
        for chunk in range(_NCH):
            if chunk + 1 < _NCH:
                _start(chunk + 1)
            _wait(chunk)
            slot = chunk % 2
            for r in range(_CH):
                zero = jnp.zeros((_V,), jnp.float32)

                def _kbody(k, accs, _slot=slot, _r=r):
                    a1, a2, a12 = accs
                    col = k * (_V * 4)
                    for u in range(4):
                        off = col + u * _V
                        va = buf1[_slot, _r, pl.ds(off, _V)]
                        vb = buf2[_slot, _r, pl.ds(off, _V)]
                        a1 = a1 + va * va
                        a2 = a2 + vb * vb
                        a12 = a12 + va * vb
                    return (a1, a2, a12)

                acc1, acc2, acc12 = lax.fori_loop(
                    0, _D // (_V * 4), _kbody, (zero, zero, zero))
                rv = (chunk * _CH + r) * _V
                res[pl.ds(rv, _V)] = acc1
                res[pl.ds(_RPW * _V + rv, _V)] = acc2
                res[pl.ds(2 * _RPW * _V + rv, _V)] = acc12
        for q in range(3):
            pltpu.sync_copy(
                res.at[pl.ds(q * _RPW * _V, _RPW * _V)],
                out_hbm.at[pl.ds(q * _SC_ROWS * _V + wid * _RPW * _V,
                                 _RPW * _V)])

    return _sc_kernel(joint1_embedding, joint2_embedding)


def _fin_body(tc_ref, s1_ref, s2_ref, s12_ref, op_ref, out_ref):
    s1 = jnp.sum(s1_ref[...], axis=1)
    s2 = jnp.sum(s2_ref[...], axis=1)
    s12 = jnp.sum(s12_ref[...], axis=1)
    op_t = op_ref[0, _TC_ROWS:_B]
    eq = (op_t == _OP_EQ).astype(jnp.float32)
    sd = s1 + s2 - 2.0 * s12
    dn = jnp.sqrt(s1) - jnp.sqrt(s2)
    eq_sd = tc_ref[0, 0] + jnp.sum(eq * sd)
    lt_sum = tc_ref[0, 1] + jnp.sum(jnp.maximum(dn, 0.0))
    gt_sum = tc_ref[0, 2] + jnp.sum(jnp.maximum(-dn, 0.0))
    eq_cnt = tc_ref[0, 3]
    has_lt = (tc_ref[0, 4] > 0.0).astype(jnp.float32)
    has_gt = (tc_ref[0, 5] > 0.0).astype(jnp.float32)
    eq_loss = eq_sd / jnp.maximum(eq_cnt * _D, 1.0)
    lt_loss = lt_sum * (1.0 / _B)
    gt_loss = gt_sum * (1.0 / _B)
    out_ref[0, 0] = (_ALPHA * eq_loss
                     + _BETA * (has_lt * lt_loss + has_gt * gt_loss))


def kernel(joint1_embedding, joint2_embedding, operation):
    op_row = operation.reshape(1, _B)
    sc_parts = _sc_rowsums(joint1_embedding, joint2_embedding)
    tc_parts = _tc_partials(joint1_embedding, joint2_embedding, operation,
                            op_row)
    sc_parts = sc_parts.reshape(3, _SC_ROWS, _V)
    out = pl.pallas_call(
        _fin_body,
        in_specs=[
            pl.BlockSpec(memory_space=pltpu.SMEM),
            pl.BlockSpec((_SC_ROWS, _V), lambda: (0, 0)),
            pl.BlockSpec((_SC_ROWS, _V), lambda: (0, 0)),
            pl.BlockSpec((_SC_ROWS, _V), lambda: (0, 0)),
            pl.BlockSpec((1, _B), lambda: (0, 0)),
        ],
        out_specs=pl.BlockSpec(memory_space=pltpu.SMEM),
        out_shape=jax.ShapeDtypeStruct((1, 1), jnp.float32),
    )(tc_parts, sc_parts[0], sc_parts[1], sc_parts[2], op_row)
    return out[0, 0]


# speedup vs baseline: 1.5440x; 1.5190x over previous
import jax
import jax.numpy as jnp
from jax.experimental import pallas as pl
from jax.experimental.pallas import tpu as pltpu

_OP_EQ, _OP_LT, _OP_GT = 0, 1, 2
_ALPHA, _BETA = 1.2, 0.7
_B, _D = 8192, 2048
_BM = 512
_NB = _B // _BM
_L = 128
_TR = 64
_NK = _D // _L
_NT = _BM // _TR


def _loss_body(op_ref, j1_ref, j2_ref, out_ref, acc_ref):
    i = pl.program_id(0)

    @pl.when(i == 0)
    def _init():
        acc_ref[...] = jnp.zeros((6, _TR, _L), jnp.float32)

    ones_b = jnp.ones((_L, _L), dtype=jnp.bfloat16)
    acc0 = acc_ref[0]
    acc1 = acc_ref[1]
    acc2 = acc_ref[2]
    for r in range(_NT):
        r0 = r * _TR
        a = j1_ref[r0:r0 + _TR, 0:_L].astype(jnp.bfloat16)
        b = j2_ref[r0:r0 + _TR, 0:_L].astype(jnp.bfloat16)
        p1 = a * a
        p2 = b * b
        p12 = a * b
        for k in range(1, _NK):
            c0 = k * _L
            a = j1_ref[r0:r0 + _TR, c0:c0 + _L].astype(jnp.bfloat16)
            b = j2_ref[r0:r0 + _TR, c0:c0 + _L].astype(jnp.bfloat16)
            p1 += a * a
            p2 += b * b
            p12 += a * b
        s1 = jax.lax.dot(p1, ones_b, preferred_element_type=jnp.float32)
        s2 = jax.lax.dot(p2, ones_b, preferred_element_type=jnp.float32)
        pd = (p1 + p2 - 2.0 * p12).astype(jnp.float32)
        op_t = op_ref[r0:r0 + _TR, :]
        eq = (op_t == _OP_EQ).astype(jnp.float32)
        lt = (op_t == _OP_LT).astype(jnp.float32)
        gt = (op_t == _OP_GT).astype(jnp.float32)
        dn = jnp.sqrt(s1) - jnp.sqrt(s2)
        acc0 = acc0 + eq * pd
        acc1 = acc1 + jnp.maximum(dn, 0.0)
        acc2 = acc2 + jnp.maximum(-dn, 0.0)
        acc_ref[3] += jnp.broadcast_to(eq, (_TR, _L))
        acc_ref[4] += jnp.broadcast_to(lt, (_TR, _L))
        acc_ref[5] += jnp.broadcast_to(gt, (_TR, _L))
    acc_ref[0] = acc0
    acc_ref[1] = acc1
    acc_ref[2] = acc2

    @pl.when(i == _NB - 1)
    def _finalize():
        inv_l = 1.0 / _L
        eq_sd = jnp.sum(acc_ref[0])
        lt_sum = jnp.sum(acc_ref[1]) * inv_l
        gt_sum = jnp.sum(acc_ref[2]) * inv_l
        eq_cnt = jnp.sum(acc_ref[3]) * inv_l
        lt_cnt = jnp.sum(acc_ref[4]) * inv_l
        gt_cnt = jnp.sum(acc_ref[5]) * inv_l
        has_lt = (lt_cnt > 0.0).astype(jnp.float32)
        has_gt = (gt_cnt > 0.0).astype(jnp.float32)
        eq_loss = eq_sd / jnp.maximum(eq_cnt * _D, 1.0)
        lt_loss = lt_sum * (1.0 / _B)
        gt_loss = gt_sum * (1.0 / _B)
        out_ref[0, 0] = (_ALPHA * eq_loss
                         + _BETA * (has_lt * lt_loss + has_gt * gt_loss))


def kernel(joint1_embedding, joint2_embedding, operation):
    out = pl.pallas_call(
        _loss_body,
        grid=(_NB,),
        in_specs=[
            pl.BlockSpec((_BM, 1), lambda i: (i, 0)),
            pl.BlockSpec((_BM, _D), lambda i: (i, 0)),
            pl.BlockSpec((_BM, _D), lambda i: (i, 0)),
        ],
        out_specs=pl.BlockSpec(memory_space=pltpu.SMEM),
        out_shape=jax.ShapeDtypeStruct((1, 1), jnp.float32),
        scratch_shapes=[
            pltpu.VMEM((6, _TR, _L), jnp.float32),
        ],
    )(operation, joint1_embedding, joint2_embedding)
    return out[0, 0]


# speedup vs baseline: 1.7694x; 1.1460x over previous
import jax
import jax.numpy as jnp
from jax.experimental import pallas as pl
from jax.experimental.pallas import tpu as pltpu

_OP_EQ, _OP_LT, _OP_GT = 0, 1, 2
_ALPHA, _BETA = 1.2, 0.7
_B, _D = 8192, 2048
_BM = 512
_NB = _B // _BM
_L = 128
_TR = 64
_NK = _D // _L
_NT = _BM // _TR


def _loss_body(op_ref, j1_ref, j2_ref, out_ref, accn_ref, acc0_ref,
               cnt_ref, pd_ref):
    i = pl.program_id(0)

    @pl.when(i == 0)
    def _init():
        accn_ref[...] = jnp.zeros((2, _TR, _L), jnp.float32)
        acc0_ref[...] = jnp.zeros((1, _L), jnp.float32)
        cnt_ref[...] = jnp.zeros((3, 1, _BM), jnp.float32)

    ones_b = jnp.ones((_L, _L), dtype=jnp.bfloat16)
    acc1 = accn_ref[0]
    acc2 = accn_ref[1]
    for r in range(_NT):
        r0 = r * _TR
        a = j1_ref[r0:r0 + _TR, 0:_L].astype(jnp.bfloat16)
        b = j2_ref[r0:r0 + _TR, 0:_L].astype(jnp.bfloat16)
        p1 = a * a
        p2 = b * b
        p12 = a * b
        for k in range(1, _NK):
            c0 = k * _L
            a = j1_ref[r0:r0 + _TR, c0:c0 + _L].astype(jnp.bfloat16)
            b = j2_ref[r0:r0 + _TR, c0:c0 + _L].astype(jnp.bfloat16)
            p1 += a * a
            p2 += b * b
            p12 += a * b
        s1 = jax.lax.dot(p1, ones_b, preferred_element_type=jnp.float32)
        s2 = jax.lax.dot(p2, ones_b, preferred_element_type=jnp.float32)
        pd_ref[r0:r0 + _TR, :] = p1 + p2 - 2.0 * p12
        dn = jnp.sqrt(s1) - jnp.sqrt(s2)
        acc1 = acc1 + jnp.maximum(dn, 0.0)
        acc2 = acc2 + jnp.maximum(-dn, 0.0)
    accn_ref[0] = acc1
    accn_ref[1] = acc2

    opv = op_ref[...]
    eqv = (opv == _OP_EQ).astype(jnp.float32)
    cnt_ref[0] += eqv
    cnt_ref[1] += (opv == _OP_LT).astype(jnp.float32)
    cnt_ref[2] += (opv == _OP_GT).astype(jnp.float32)
    acc0_ref[...] += jax.lax.dot(eqv.astype(jnp.bfloat16), pd_ref[...],
                                 preferred_element_type=jnp.float32)

    @pl.when(i == _NB - 1)
    def _finalize():
        inv_l = 1.0 / _L
        eq_sd = jnp.sum(acc0_ref[...])
        lt_sum = jnp.sum(accn_ref[0]) * inv_l
        gt_sum = jnp.sum(accn_ref[1]) * inv_l
        eq_cnt = jnp.sum(cnt_ref[0])
        lt_cnt = jnp.sum(cnt_ref[1])
        gt_cnt = jnp.sum(cnt_ref[2])
        has_lt = (lt_cnt > 0.0).astype(jnp.float32)
        has_gt = (gt_cnt > 0.0).astype(jnp.float32)
        eq_loss = eq_sd / jnp.maximum(eq_cnt * _D, 1.0)
        lt_loss = lt_sum * (1.0 / _B)
        gt_loss = gt_sum * (1.0 / _B)
        out_ref[0, 0] = (_ALPHA * eq_loss
                         + _BETA * (has_lt * lt_loss + has_gt * gt_loss))


def kernel(joint1_embedding, joint2_embedding, operation):
    op_row = operation.reshape(1, _B)
    out = pl.pallas_call(
        _loss_body,
        grid=(_NB,),
        in_specs=[
            pl.BlockSpec((1, _BM), lambda i: (0, i)),
            pl.BlockSpec((_BM, _D), lambda i: (i, 0)),
            pl.BlockSpec((_BM, _D), lambda i: (i, 0)),
        ],
        out_specs=pl.BlockSpec(memory_space=pltpu.SMEM),
        out_shape=jax.ShapeDtypeStruct((1, 1), jnp.float32),
        scratch_shapes=[
            pltpu.VMEM((2, _TR, _L), jnp.float32),
            pltpu.VMEM((1, _L), jnp.float32),
            pltpu.VMEM((3, 1, _BM), jnp.float32),
            pltpu.VMEM((_BM, _L), jnp.bfloat16),
        ],
    )(op_row, joint1_embedding, joint2_embedding)
    return out[0, 0]
